# baseline (speedup 1.0000x reference)
import jax
import jax.numpy as jnp
from jax.experimental import pallas as pl
from jax.experimental.pallas import tpu as pltpu

_BM = 200


def _gcn_body(x_ref, a1_ref, a2_ref, o_ref, xhl_ref):
    d = x_ref.shape[1]

    @pl.when(pl.program_id(0) == 0)
    def _prep():
        xf = x_ref[...]
        xh = xf.astype(jnp.bfloat16)
        xl = (xf - xh.astype(jnp.float32)).astype(jnp.bfloat16)
        xhl_ref[:, :d] = xh
        xhl_ref[:, d:] = xl

    xhl = xhl_ref[...]
    a1 = a1_ref[...].astype(jnp.bfloat16)
    a2 = a2_ref[...].astype(jnp.bfloat16)
    p1 = jnp.dot(a1, xhl, preferred_element_type=jnp.float32)
    p2 = jnp.dot(a2, xhl, preferred_element_type=jnp.float32)
    o_ref[:, :d] = p1[:, :d] + p1[:, d:]
    o_ref[:, d:] = p2[:, :d] + p2[:, d:]


@jax.jit
def kernel(x, adj_t, adj_t2):
    n, d = x.shape
    grid = (n // _BM,)
    return pl.pallas_call(
        _gcn_body,
        grid=grid,
        in_specs=[
            pl.BlockSpec((n, d), lambda i: (0, 0)),
            pl.BlockSpec((_BM, n), lambda i: (i, 0)),
            pl.BlockSpec((_BM, n), lambda i: (i, 0)),
        ],
        out_specs=pl.BlockSpec((_BM, 2 * d), lambda i: (i, 0)),
        out_shape=jax.ShapeDtypeStruct((n, 2 * d), jnp.float32),
        scratch_shapes=[pltpu.VMEM((n, 2 * d), jnp.bfloat16)],
        compiler_params=pltpu.CompilerParams(
            dimension_semantics=("arbitrary",),
        ),
    )(x, adj_t, adj_t2)


# speedup vs baseline: 1.0142x; 1.0142x over previous
import jax
import jax.numpy as jnp
from jax.experimental import pallas as pl
from jax.experimental.pallas import tpu as pltpu

_BM = 200


def _gcn_body(x_ref, a1_ref, a2_ref, o_ref):
    d = x_ref.shape[1]
    xf = x_ref[...]
    p1 = jnp.dot(a1_ref[...], xf, preferred_element_type=jnp.float32)
    p2 = jnp.dot(a2_ref[...], xf, preferred_element_type=jnp.float32)
    o_ref[:, :d] = p1
    o_ref[:, d:] = p2


@jax.jit
def kernel(x, adj_t, adj_t2):
    n, d = x.shape
    grid = (n // _BM,)
    return pl.pallas_call(
        _gcn_body,
        grid=grid,
        in_specs=[
            pl.BlockSpec((n, d), lambda i: (0, 0)),
            pl.BlockSpec((_BM, n), lambda i: (i, 0)),
            pl.BlockSpec((_BM, n), lambda i: (i, 0)),
        ],
        out_specs=pl.BlockSpec((_BM, 2 * d), lambda i: (i, 0)),
        out_shape=jax.ShapeDtypeStruct((n, 2 * d), jnp.float32),
        compiler_params=pltpu.CompilerParams(
            dimension_semantics=("arbitrary",),
        ),
    )(x, adj_t, adj_t2)
